# speedup vs baseline: 2.1949x; 2.1949x over previous
import functools

import jax
import jax.numpy as jnp
from jax import lax
from jax.experimental import pallas as pl
from jax.experimental.pallas import tpu as pltpu
from jax.experimental.pallas import tpu_sc as plsc

_CHUNK = 128


@functools.cache
def _build(n_rows: int, d: int, n_bins: int):
  info = plsc.get_sparse_core_info()
  nw = info.num_cores * info.num_subcores
  assert n_rows % (nw * _CHUNK) == 0
  per_w = n_rows // nw
  n_chunks = per_w // _CHUNK

  mesh = plsc.VectorSubcoreMesh(core_axis_name="c", subcore_axis_name="s")

  @functools.partial(
      pl.kernel,
      mesh=mesh,
      out_type=jax.ShapeDtypeStruct((n_rows, d), jnp.float32),
      scratch_types=[
          pltpu.VMEM((_CHUNK,), jnp.int32),
          pltpu.VMEM((_CHUNK, d), jnp.float32),
          pltpu.SemaphoreType.DMA,
      ],
  )
  def emb(ids_hbm, table_hbm, out_hbm, idx_v, rows_v, sem):
    wid = lax.axis_index("s") * info.num_cores + lax.axis_index("c")
    base = wid * per_w

    def chunk(i, carry):
      off = base + i * _CHUNK
      pltpu.sync_copy(ids_hbm.at[pl.ds(off, _CHUNK)], idx_v)
      pltpu.async_copy(table_hbm.at[idx_v], rows_v, sem).wait()
      pltpu.sync_copy(rows_v, out_hbm.at[pl.ds(off, _CHUNK)])
      return carry

    lax.fori_loop(0, n_chunks, chunk, 0)

  return emb


def kernel(value_ids, value_floats, bin_emb_weight):
  del value_floats
  b, l = value_ids.shape
  n_bins, d = bin_emb_weight.shape
  ids_flat = value_ids.reshape(-1).astype(jnp.int32)
  emb = _build(b * l, d, n_bins)
  out = emb(ids_flat, bin_emb_weight.astype(jnp.float32))
  return out.reshape(b, l, d)


# speedup vs baseline: 2.2210x; 1.0119x over previous
import functools

import jax
import jax.numpy as jnp
from jax import lax
from jax.experimental import pallas as pl
from jax.experimental.pallas import tpu as pltpu
from jax.experimental.pallas import tpu_sc as plsc

_CHUNK = 128
_K = 2
_GROUP = _K * _CHUNK


@functools.cache
def _build(n_rows: int, d: int):
  info = plsc.get_sparse_core_info()
  nw = info.num_cores * info.num_subcores
  assert n_rows % (nw * 2 * _GROUP) == 0
  per_w = n_rows // nw
  n_groups = per_w // _GROUP
  id_rows_w = per_w // _CHUNK

  mesh = plsc.VectorSubcoreMesh(core_axis_name="c", subcore_axis_name="s")

  @functools.partial(
      pl.kernel,
      mesh=mesh,
      out_type=jax.ShapeDtypeStruct((n_rows, d), jnp.float32),
      scratch_types=[
          pltpu.VMEM((_K, _CHUNK), jnp.int32),
          pltpu.VMEM((_K, _CHUNK), jnp.int32),
          pltpu.VMEM((_GROUP, d), jnp.float32),
          pltpu.VMEM((_GROUP, d), jnp.float32),
          pltpu.SemaphoreType.DMA,
          pltpu.SemaphoreType.DMA,
          pltpu.SemaphoreType.DMA,
          pltpu.SemaphoreType.DMA,
          pltpu.SemaphoreType.DMA,
          pltpu.SemaphoreType.DMA,
      ],
  )
  def emb(ids_hbm, table_hbm, out_hbm, idx0, idx1, rows0, rows1,
          sidx0, sidx1, sg0, sg1, ss0, ss1):
    wid = lax.axis_index("s") * info.num_cores + lax.axis_index("c")
    out_base = wid * per_w
    id_base = wid * id_rows_w
    last_id_row = id_base + id_rows_w - _K

    def idx_start(dst, sem, id_row):
      r = jnp.minimum(id_row, last_id_row)
      pltpu.async_copy(ids_hbm.at[pl.ds(r, _K)], dst, sem)

    def idx_wait(dst, sem):
      pltpu.make_async_copy(ids_hbm.at[pl.ds(id_base, _K)], dst, sem).wait()

    def fire_gathers(idx, rows, sem):
      return [
          pltpu.async_copy(table_hbm.at[idx.at[j]],
                           rows.at[pl.ds(j * _CHUNK, _CHUNK)], sem)
          for j in range(_K)
      ]

    def scatter_start(rows, off, sem):
      pltpu.async_copy(rows, out_hbm.at[pl.ds(off, _GROUP)], sem)

    def scatter_wait(rows, sem):
      pltpu.make_async_copy(rows, out_hbm.at[pl.ds(out_base, _GROUP)], sem).wait()

    def body(t, carry, first):
      g_even = 2 * t
      off_e = out_base + g_even * _GROUP
      if not first:
        scatter_wait(rows0, ss0)
      ge = fire_gathers(idx0, rows0, sg0)
      if not first:
        scatter_wait(rows1, ss1)
      idx_wait(idx1, sidx1)
      for c in ge:
        c.wait()
      idx_start(idx0, sidx0, id_base + (g_even + 2) * _K)
      go = fire_gathers(idx1, rows1, sg1)
      scatter_start(rows0, off_e, ss0)
      for c in go:
        c.wait()
      idx_start(idx1, sidx1, id_base + (g_even + 3) * _K)
      scatter_start(rows1, off_e + _GROUP, ss1)
      idx_wait(idx0, sidx0)
      return carry

    pltpu.sync_copy(ids_hbm.at[pl.ds(id_base, _K)], idx0)
    pltpu.async_copy(ids_hbm.at[pl.ds(id_base + _K, _K)], idx1, sidx1)
    body(0, 0, first=True)
    lax.fori_loop(1, n_groups // 2,
                  functools.partial(body, first=False), 0)
    scatter_wait(rows0, ss0)
    scatter_wait(rows1, ss1)
    idx_wait(idx1, sidx1)

  return emb


def kernel(value_ids, value_floats, bin_emb_weight):
  del value_floats
  b, l = value_ids.shape
  n_bins, d = bin_emb_weight.shape
  n_rows = b * l
  ids2d = value_ids.reshape(n_rows // _CHUNK, _CHUNK).astype(jnp.int32)
  emb = _build(n_rows, d)
  out = emb(ids2d, bin_emb_weight.astype(jnp.float32))
  return out.reshape(b, l, d)


# speedup vs baseline: 19.0231x; 8.5651x over previous
import functools

import jax
import jax.numpy as jnp
from jax import lax
from jax.experimental import pallas as pl
from jax.experimental.pallas import tpu as pltpu
from jax.experimental.pallas import tpu_sc as plsc

_CHUNK = 128
_K = 2
_GROUP = _K * _CHUNK


@functools.cache
def _build(n_rows: int, d: int, n_bins: int):
  info = plsc.get_sparse_core_info()
  nw = info.num_cores * info.num_subcores
  assert n_rows % (nw * 2 * _GROUP) == 0
  per_w = n_rows // nw
  n_groups = per_w // _GROUP
  id_rows_w = per_w // _CHUNK

  mesh = plsc.VectorSubcoreMesh(core_axis_name="c", subcore_axis_name="s")

  @functools.partial(
      pl.kernel,
      mesh=mesh,
      out_type=jax.ShapeDtypeStruct((n_rows, d), jnp.float32),
      scratch_types=[
          pltpu.VMEM((_K, _CHUNK), jnp.int32),
          pltpu.VMEM((_K, _CHUNK), jnp.int32),
          pltpu.VMEM((_GROUP, d), jnp.float32),
          pltpu.VMEM((_GROUP, d), jnp.float32),
          pltpu.VMEM_SHARED((n_bins, d), jnp.float32),
          pltpu.SemaphoreType.DMA,
          pltpu.SemaphoreType.DMA,
          pltpu.SemaphoreType.DMA,
          pltpu.SemaphoreType.DMA,
          pltpu.SemaphoreType.DMA,
          pltpu.SemaphoreType.DMA,
      ],
  )
  def emb(ids_hbm, table_hbm, out_hbm, idx0, idx1, rows0, rows1, table_sh,
          sidx0, sidx1, sg0, sg1, ss0, ss1):
    @pl.when(lax.axis_index("s") == 0)
    def _stage_table():
      pltpu.sync_copy(table_hbm, table_sh)
    plsc.subcore_barrier()

    wid = lax.axis_index("s") * info.num_cores + lax.axis_index("c")
    out_base = wid * per_w
    id_base = wid * id_rows_w
    last_id_row = id_base + id_rows_w - _K

    def idx_start(dst, sem, id_row):
      r = jnp.minimum(id_row, last_id_row)
      pltpu.async_copy(ids_hbm.at[pl.ds(r, _K)], dst, sem)

    def idx_wait(dst, sem):
      pltpu.make_async_copy(ids_hbm.at[pl.ds(id_base, _K)], dst, sem).wait()

    def fire_gathers(idx, rows, sem):
      return [
          pltpu.async_copy(table_sh.at[idx.at[j]],
                           rows.at[pl.ds(j * _CHUNK, _CHUNK)], sem)
          for j in range(_K)
      ]

    def scatter_start(rows, off, sem):
      pltpu.async_copy(rows, out_hbm.at[pl.ds(off, _GROUP)], sem)

    def scatter_wait(rows, sem):
      pltpu.make_async_copy(rows, out_hbm.at[pl.ds(out_base, _GROUP)], sem).wait()

    def body(t, carry, first):
      g_even = 2 * t
      off_e = out_base + g_even * _GROUP
      if not first:
        scatter_wait(rows0, ss0)
      ge = fire_gathers(idx0, rows0, sg0)
      if not first:
        scatter_wait(rows1, ss1)
      idx_wait(idx1, sidx1)
      for c in ge:
        c.wait()
      idx_start(idx0, sidx0, id_base + (g_even + 2) * _K)
      go = fire_gathers(idx1, rows1, sg1)
      scatter_start(rows0, off_e, ss0)
      for c in go:
        c.wait()
      idx_start(idx1, sidx1, id_base + (g_even + 3) * _K)
      scatter_start(rows1, off_e + _GROUP, ss1)
      idx_wait(idx0, sidx0)
      return carry

    pltpu.sync_copy(ids_hbm.at[pl.ds(id_base, _K)], idx0)
    pltpu.async_copy(ids_hbm.at[pl.ds(id_base + _K, _K)], idx1, sidx1)
    body(0, 0, first=True)
    lax.fori_loop(1, n_groups // 2,
                  functools.partial(body, first=False), 0)
    scatter_wait(rows0, ss0)
    scatter_wait(rows1, ss1)
    idx_wait(idx1, sidx1)

  return emb


def kernel(value_ids, value_floats, bin_emb_weight):
  del value_floats
  b, l = value_ids.shape
  n_bins, d = bin_emb_weight.shape
  n_rows = b * l
  ids2d = value_ids.reshape(n_rows // _CHUNK, _CHUNK).astype(jnp.int32)
  emb = _build(n_rows, d, n_bins)
  out = emb(ids2d, bin_emb_weight.astype(jnp.float32))
  return out.reshape(b, l, d)
